# baseline (speedup 1.0000x reference)
import functools

import jax
import jax.numpy as jnp
from jax import lax
from jax.experimental import pallas as pl
from jax.experimental.pallas import tpu as pltpu
from jax.experimental.pallas import tpu_sc as plsc

NE = 8
NTOPK = 2
DM = 1024
DFF = 4096

T = 128
FT = 1024
NF = DFF // FT
NT_LANES = 128


def _routing_body(x_ref, rwt_ref, w0_ref, w1_ref, w0b_ref, w1b_ref,
                  p0_ref, p1_ref, te_ref):
    m_rows = x_ref.shape[0]
    x = x_ref[...]
    rwt = rwt_ref[...]
    logits = jnp.dot(x, rwt, preferred_element_type=jnp.float32)
    mx = jnp.max(logits, axis=1, keepdims=True)
    ex = jnp.exp(logits - mx)
    probs = ex / jnp.sum(ex, axis=1, keepdims=True)
    lane = lax.broadcasted_iota(jnp.int32, probs.shape, 1)
    m1 = jnp.max(probs, axis=1, keepdims=True)
    i1 = jnp.min(jnp.where(probs == m1, lane, NE), axis=1, keepdims=True)
    masked = jnp.where(lane == i1, -1.0, probs)
    m2 = jnp.max(masked, axis=1, keepdims=True)
    i2 = jnp.min(jnp.where(masked == m2, lane, NE), axis=1, keepdims=True)
    match0 = (lane == i1).astype(jnp.float32)
    match1 = (lane == i2).astype(jnp.float32)

    w0_ref[...] = m1
    w1_ref[...] = m2
    w0b_ref[...] = jnp.broadcast_to(m1, (m_rows, 16))
    w1b_ref[...] = jnp.broadcast_to(m2, (m_rows, 16))

    ri = lax.broadcasted_iota(jnp.int32, (m_rows, m_rows), 0)
    ci = lax.broadcasted_iota(jnp.int32, (m_rows, m_rows), 1)
    ltri = (ci < ri).astype(jnp.float32)
    a0 = jnp.dot(ltri, match0, preferred_element_type=jnp.float32)
    a1 = jnp.dot(ltri, match1, preferred_element_type=jnp.float32)

    counts = jnp.sum(match0 + match1, axis=0, keepdims=True)
    padded = jnp.ceil(counts / T) * T
    elane_r = lax.broadcasted_iota(jnp.int32, (NE, NE), 0)
    elane_c = lax.broadcasted_iota(jnp.int32, (NE, NE), 1)
    excl = (elane_r < elane_c).astype(jnp.float32)
    off = jnp.dot(padded, excl, preferred_element_type=jnp.float32)

    pos = off + a0 + a1
    p0_ref[...] = jnp.sum(match0 * pos, axis=1, keepdims=True).astype(jnp.int32)
    p1_ref[...] = jnp.sum(match1 * pos, axis=1, keepdims=True).astype(jnp.int32)

    ts = (lax.broadcasted_iota(jnp.int32, (NT_LANES, NE), 0) * T).astype(jnp.float32)
    offb = jnp.broadcast_to(off, (NT_LANES, NE))
    pcb = jnp.broadcast_to(padded, (NT_LANES, NE))
    inb = jnp.logical_and(ts >= offb, ts < offb + pcb)
    eid = lax.broadcasted_iota(jnp.int32, (NT_LANES, NE), 1)
    te_ref[...] = jnp.sum(jnp.where(inb, eid + 1, 0), axis=1, keepdims=True) - 1


def _routing_call(xf, rwt):
    m_rows = xf.shape[0]
    return pl.pallas_call(
        _routing_body,
        out_shape=[
            jax.ShapeDtypeStruct((m_rows, 1), jnp.float32),
            jax.ShapeDtypeStruct((m_rows, 1), jnp.float32),
            jax.ShapeDtypeStruct((m_rows, 16), jnp.float32),
            jax.ShapeDtypeStruct((m_rows, 16), jnp.float32),
            jax.ShapeDtypeStruct((m_rows, 1), jnp.int32),
            jax.ShapeDtypeStruct((m_rows, 1), jnp.int32),
            jax.ShapeDtypeStruct((NT_LANES, 1), jnp.int32),
        ],
    )(xf, rwt)


def _gemm_body(te_ref, x_ref, w1g_ref, w1u_ref, w2_ref, out_ref, acc_ref):
    j = pl.program_id(0)
    i = pl.program_id(1)
    e = te_ref[i]
    sl = pl.ds(i * T, T)

    @pl.when(e >= 0)
    def _():
        x = x_ref[...].astype(jnp.bfloat16)
        g = jnp.dot(x, w1g_ref[0], preferred_element_type=jnp.float32)
        u = jnp.dot(x, w1u_ref[0], preferred_element_type=jnp.float32)
        act = (g * jax.nn.sigmoid(g) * u).astype(jnp.bfloat16)
        part = jnp.dot(act, w2_ref[0], preferred_element_type=jnp.float32)

        @pl.when(j == 0)
        def _():
            acc_ref[sl, :] = part

        @pl.when(jnp.logical_and(j > 0, j < NF - 1))
        def _():
            acc_ref[sl, :] += part

        @pl.when(j == NF - 1)
        def _():
            out_ref[...] = acc_ref[sl, :] + part

    @pl.when(jnp.logical_and(e < 0, j == NF - 1))
    def _():
        out_ref[...] = jnp.zeros_like(out_ref)


def _gemm_call(xs, te, w1, w2):
    rmax = xs.shape[0]
    nt = rmax // T
    grid_spec = pltpu.PrefetchScalarGridSpec(
        num_scalar_prefetch=1,
        grid=(NF, nt),
        in_specs=[
            pl.BlockSpec((T, DM), lambda j, i, te_s: (i, 0)),
            pl.BlockSpec((1, DM, FT),
                         lambda j, i, te_s: (jnp.maximum(te_s[i], 0), 0, j)),
            pl.BlockSpec((1, DM, FT),
                         lambda j, i, te_s: (jnp.maximum(te_s[i], 0), 0, j + NF)),
            pl.BlockSpec((1, FT, DM),
                         lambda j, i, te_s: (jnp.maximum(te_s[i], 0), j, 0)),
        ],
        out_specs=pl.BlockSpec(
            (T, DM), lambda j, i, te_s: (jnp.where(j == NF - 1, i, 0), 0)),
        scratch_shapes=[pltpu.VMEM((rmax, DM), jnp.float32)],
    )
    w1h = w1.astype(jnp.bfloat16)
    w2h = w2.astype(jnp.bfloat16)
    return pl.pallas_call(
        _gemm_body,
        grid_spec=grid_spec,
        out_shape=jax.ShapeDtypeStruct((rmax, DM), jnp.float32),
        compiler_params=pltpu.CompilerParams(
            dimension_semantics=("arbitrary", "arbitrary")
        ),
    )(te, xs, w1h, w1h, w2h)


def _scatter_call(xf, p0, p1, rmax):
    m_rows = xf.shape[0]
    n_per = m_rows // 32
    nchunks = n_per // 16
    mesh = plsc.VectorSubcoreMesh(core_axis_name="c", subcore_axis_name="s")

    @functools.partial(
        pl.kernel,
        mesh=mesh,
        out_type=jax.ShapeDtypeStruct((rmax, DM), jnp.float32),
        scratch_types=[
            pltpu.VMEM((16, DM), jnp.float32),
            pltpu.VMEM((n_per,), jnp.int32),
            pltpu.VMEM((n_per,), jnp.int32),
            pltpu.SemaphoreType.DMA,
        ],
    )
    def _scatter_k(x_hbm, p0_hbm, p1_hbm, xs_hbm, xbuf, p0v, p1v, sem):
        wid = lax.axis_index("s") * 2 + lax.axis_index("c")
        base = wid * n_per
        pltpu.sync_copy(p0_hbm.at[pl.ds(base, n_per)], p0v)
        pltpu.sync_copy(p1_hbm.at[pl.ds(base, n_per)], p1v)
        for c in range(nchunks):
            pltpu.sync_copy(x_hbm.at[pl.ds(base + c * 16, 16)], xbuf)
            pltpu.async_copy(xbuf, xs_hbm.at[p0v[pl.ds(c * 16, 16)]], sem).wait()
            pltpu.async_copy(xbuf, xs_hbm.at[p1v[pl.ds(c * 16, 16)]], sem).wait()

    return _scatter_k(xf, p0, p1)


def _combine_call(ys, p0, p1, w0b, w1b):
    m_rows = w0b.shape[0]
    n_per = m_rows // 32
    nchunks = n_per // 16
    mesh = plsc.VectorSubcoreMesh(core_axis_name="c", subcore_axis_name="s")

    @functools.partial(
        pl.kernel,
        mesh=mesh,
        out_type=jax.ShapeDtypeStruct((m_rows, DM), jnp.float32),
        scratch_types=[
            pltpu.VMEM((16, DM), jnp.float32),
            pltpu.VMEM((16, DM), jnp.float32),
            pltpu.VMEM((16, DM), jnp.float32),
            pltpu.VMEM((n_per,), jnp.int32),
            pltpu.VMEM((n_per,), jnp.int32),
            pltpu.VMEM((16, 16), jnp.float32),
            pltpu.VMEM((16, 16), jnp.float32),
            pltpu.SemaphoreType.DMA,
            pltpu.SemaphoreType.DMA,
        ],
    )
    def _combine_k(y_hbm, p0_hbm, p1_hbm, w0b_hbm, w1b_hbm, out_hbm,
                   y0buf, y1buf, obuf, p0v, p1v, w0v, w1v, sem0, sem1):
        wid = lax.axis_index("s") * 2 + lax.axis_index("c")
        base = wid * n_per
        pltpu.sync_copy(p0_hbm.at[pl.ds(base, n_per)], p0v)
        pltpu.sync_copy(p1_hbm.at[pl.ds(base, n_per)], p1v)
        for c in range(nchunks):
            cb = base + c * 16
            cp0 = pltpu.async_copy(y_hbm.at[p0v[pl.ds(c * 16, 16)]], y0buf, sem0)
            cp1 = pltpu.async_copy(y_hbm.at[p1v[pl.ds(c * 16, 16)]], y1buf, sem1)
            pltpu.sync_copy(w0b_hbm.at[pl.ds(cb, 16)], w0v)
            pltpu.sync_copy(w1b_hbm.at[pl.ds(cb, 16)], w1v)
            cp0.wait()
            cp1.wait()

            def _row(r, _):
                wr0 = w0v[r]
                wr1 = w1v[r]
                for k in range(DM // 16):
                    sl = pl.ds(k * 16, 16)
                    obuf[r, sl] = wr0 * y0buf[r, sl] + wr1 * y1buf[r, sl]
                return 0

            lax.fori_loop(0, 16, _row, 0)
            pltpu.sync_copy(obuf, out_hbm.at[pl.ds(cb, 16)])

    return _combine_k(ys, p0, p1, w0b, w1b)


def kernel(x, router_w, w1, w2):
    xf = x.reshape(-1, DM)
    m_rows = xf.shape[0]
    rmax = NTOPK * m_rows + NE * T
    rwt = router_w.T

    tw0, tw1, w0b, w1b, p0, p1, te = _routing_call(xf, rwt)
    p0f = p0.reshape(-1)
    p1f = p1.reshape(-1)

    xs = _scatter_call(xf, p0f, p1f, rmax)
    ys = _gemm_call(xs, te.reshape(-1), w1, w2)
    out = _combine_call(ys, p0f, p1f, w0b, w1b)

    topk_weights = jnp.concatenate([tw0, tw1], axis=1)
    return out.reshape(x.shape), topk_weights


# speedup vs baseline: 1.4128x; 1.4128x over previous
import functools

import jax
import jax.numpy as jnp
from jax import lax
from jax.experimental import pallas as pl
from jax.experimental.pallas import tpu as pltpu
from jax.experimental.pallas import tpu_sc as plsc

NE = 8
NTOPK = 2
DM = 1024
DFF = 4096

T = 256
FT = 1024
NF = DFF // FT
NT_LANES = 128


def _routing_body(x_ref, rwt_ref, w0_ref, w1_ref, w0b_ref, w1b_ref,
                  p0_ref, p1_ref, te_ref):
    m_rows = x_ref.shape[0]
    x = x_ref[...]
    rwt = rwt_ref[...]
    logits = jnp.dot(x, rwt, preferred_element_type=jnp.float32)
    mx = jnp.max(logits, axis=1, keepdims=True)
    ex = jnp.exp(logits - mx)
    probs = ex / jnp.sum(ex, axis=1, keepdims=True)
    lane = lax.broadcasted_iota(jnp.int32, probs.shape, 1)
    m1 = jnp.max(probs, axis=1, keepdims=True)
    i1 = jnp.min(jnp.where(probs == m1, lane, NE), axis=1, keepdims=True)
    masked = jnp.where(lane == i1, -1.0, probs)
    m2 = jnp.max(masked, axis=1, keepdims=True)
    i2 = jnp.min(jnp.where(masked == m2, lane, NE), axis=1, keepdims=True)
    match0 = (lane == i1).astype(jnp.float32)
    match1 = (lane == i2).astype(jnp.float32)

    w0_ref[...] = m1
    w1_ref[...] = m2
    w0b_ref[...] = jnp.broadcast_to(m1, (m_rows, 16))
    w1b_ref[...] = jnp.broadcast_to(m2, (m_rows, 16))

    ri = lax.broadcasted_iota(jnp.int32, (m_rows, m_rows), 0)
    ci = lax.broadcasted_iota(jnp.int32, (m_rows, m_rows), 1)
    ltri = (ci < ri).astype(jnp.float32)
    a0 = jnp.dot(ltri, match0, preferred_element_type=jnp.float32)
    a1 = jnp.dot(ltri, match1, preferred_element_type=jnp.float32)

    counts = jnp.sum(match0 + match1, axis=0, keepdims=True)
    padded = jnp.ceil(counts / T) * T
    elane_r = lax.broadcasted_iota(jnp.int32, (NE, NE), 0)
    elane_c = lax.broadcasted_iota(jnp.int32, (NE, NE), 1)
    excl = (elane_r < elane_c).astype(jnp.float32)
    off = jnp.dot(padded, excl, preferred_element_type=jnp.float32)

    pos = off + a0 + a1
    p0_ref[...] = jnp.sum(match0 * pos, axis=1, keepdims=True).astype(jnp.int32)
    p1_ref[...] = jnp.sum(match1 * pos, axis=1, keepdims=True).astype(jnp.int32)

    ts = (lax.broadcasted_iota(jnp.int32, (NT_LANES, NE), 0) * T).astype(jnp.float32)
    offb = jnp.broadcast_to(off, (NT_LANES, NE))
    pcb = jnp.broadcast_to(padded, (NT_LANES, NE))
    inb = jnp.logical_and(ts >= offb, ts < offb + pcb)
    eid = lax.broadcasted_iota(jnp.int32, (NT_LANES, NE), 1)
    te_ref[...] = jnp.sum(jnp.where(inb, eid + 1, 0), axis=1, keepdims=True) - 1


def _routing_call(xf, rwt):
    m_rows = xf.shape[0]
    return pl.pallas_call(
        _routing_body,
        out_shape=[
            jax.ShapeDtypeStruct((m_rows, 1), jnp.float32),
            jax.ShapeDtypeStruct((m_rows, 1), jnp.float32),
            jax.ShapeDtypeStruct((m_rows, 16), jnp.float32),
            jax.ShapeDtypeStruct((m_rows, 16), jnp.float32),
            jax.ShapeDtypeStruct((m_rows, 1), jnp.int32),
            jax.ShapeDtypeStruct((m_rows, 1), jnp.int32),
            jax.ShapeDtypeStruct((NT_LANES, 1), jnp.int32),
        ],
    )(xf, rwt)


def _gemm_body(te_ref, x_ref, w1g_ref, w1u_ref, w2_ref, out_ref, acc_ref):
    j = pl.program_id(0)
    i = pl.program_id(1)
    e = te_ref[i]
    sl = pl.ds(i * T, T)

    @pl.when(e >= 0)
    def _():
        x = x_ref[...]
        g = jnp.dot(x, w1g_ref[0], preferred_element_type=jnp.float32)
        u = jnp.dot(x, w1u_ref[0], preferred_element_type=jnp.float32)
        act = g * jax.nn.sigmoid(g) * u
        part = jnp.dot(act, w2_ref[0], preferred_element_type=jnp.float32)

        @pl.when(j == 0)
        def _():
            acc_ref[sl, :] = part

        @pl.when(jnp.logical_and(j > 0, j < NF - 1))
        def _():
            acc_ref[sl, :] += part

        @pl.when(j == NF - 1)
        def _():
            out_ref[...] = acc_ref[sl, :] + part

    @pl.when(jnp.logical_and(e < 0, j == NF - 1))
    def _():
        out_ref[...] = jnp.zeros_like(out_ref)


def _gemm_call(xs, te, w1, w2):
    rmax = xs.shape[0]
    nt = rmax // T
    grid_spec = pltpu.PrefetchScalarGridSpec(
        num_scalar_prefetch=1,
        grid=(NF, nt),
        in_specs=[
            pl.BlockSpec((T, DM), lambda j, i, te_s: (i, 0)),
            pl.BlockSpec((1, DM, FT),
                         lambda j, i, te_s: (jnp.maximum(te_s[i], 0), 0, j)),
            pl.BlockSpec((1, DM, FT),
                         lambda j, i, te_s: (jnp.maximum(te_s[i], 0), 0, j + NF)),
            pl.BlockSpec((1, FT, DM),
                         lambda j, i, te_s: (jnp.maximum(te_s[i], 0), j, 0)),
        ],
        out_specs=pl.BlockSpec(
            (T, DM), lambda j, i, te_s: (jnp.where(j == NF - 1, i, 0), 0)),
        scratch_shapes=[pltpu.VMEM((rmax, DM), jnp.float32)],
    )
    return pl.pallas_call(
        _gemm_body,
        grid_spec=grid_spec,
        out_shape=jax.ShapeDtypeStruct((rmax, DM), jnp.float32),
        compiler_params=pltpu.CompilerParams(
            dimension_semantics=("arbitrary", "arbitrary")
        ),
    )(te, xs, w1, w1, w2)


def _scatter_call(xf, p0, p1, rmax):
    m_rows = xf.shape[0]
    n_per = m_rows // 32
    nchunks = n_per // 16
    mesh = plsc.VectorSubcoreMesh(core_axis_name="c", subcore_axis_name="s")

    @functools.partial(
        pl.kernel,
        mesh=mesh,
        out_type=jax.ShapeDtypeStruct((rmax, DM), jnp.float32),
        scratch_types=[
            pltpu.VMEM((16, DM), jnp.float32),
            pltpu.VMEM((n_per,), jnp.int32),
            pltpu.VMEM((n_per,), jnp.int32),
            pltpu.SemaphoreType.DMA,
        ],
    )
    def _scatter_k(x_hbm, p0_hbm, p1_hbm, xs_hbm, xbuf, p0v, p1v, sem):
        wid = lax.axis_index("s") * 2 + lax.axis_index("c")
        base = wid * n_per
        pltpu.sync_copy(p0_hbm.at[pl.ds(base, n_per)], p0v)
        pltpu.sync_copy(p1_hbm.at[pl.ds(base, n_per)], p1v)
        for c in range(nchunks):
            pltpu.sync_copy(x_hbm.at[pl.ds(base + c * 16, 16)], xbuf)
            pltpu.async_copy(xbuf, xs_hbm.at[p0v[pl.ds(c * 16, 16)]], sem).wait()
            pltpu.async_copy(xbuf, xs_hbm.at[p1v[pl.ds(c * 16, 16)]], sem).wait()

    return _scatter_k(xf, p0, p1)


def _combine_call(ys, p0, p1, w0b, w1b):
    m_rows = w0b.shape[0]
    n_per = m_rows // 32
    nchunks = n_per // 16
    mesh = plsc.VectorSubcoreMesh(core_axis_name="c", subcore_axis_name="s")

    @functools.partial(
        pl.kernel,
        mesh=mesh,
        out_type=jax.ShapeDtypeStruct((m_rows, DM), jnp.float32),
        scratch_types=[
            pltpu.VMEM((16, DM), jnp.float32),
            pltpu.VMEM((16, DM), jnp.float32),
            pltpu.VMEM((16, DM), jnp.float32),
            pltpu.VMEM((n_per,), jnp.int32),
            pltpu.VMEM((n_per,), jnp.int32),
            pltpu.VMEM((16, 16), jnp.float32),
            pltpu.VMEM((16, 16), jnp.float32),
            pltpu.SemaphoreType.DMA,
            pltpu.SemaphoreType.DMA,
        ],
    )
    def _combine_k(y_hbm, p0_hbm, p1_hbm, w0b_hbm, w1b_hbm, out_hbm,
                   y0buf, y1buf, obuf, p0v, p1v, w0v, w1v, sem0, sem1):
        wid = lax.axis_index("s") * 2 + lax.axis_index("c")
        base = wid * n_per
        pltpu.sync_copy(p0_hbm.at[pl.ds(base, n_per)], p0v)
        pltpu.sync_copy(p1_hbm.at[pl.ds(base, n_per)], p1v)
        for c in range(nchunks):
            cb = base + c * 16
            cp0 = pltpu.async_copy(y_hbm.at[p0v[pl.ds(c * 16, 16)]], y0buf, sem0)
            cp1 = pltpu.async_copy(y_hbm.at[p1v[pl.ds(c * 16, 16)]], y1buf, sem1)
            pltpu.sync_copy(w0b_hbm.at[pl.ds(cb, 16)], w0v)
            pltpu.sync_copy(w1b_hbm.at[pl.ds(cb, 16)], w1v)
            cp0.wait()
            cp1.wait()

            def _row(r, _):
                wr0 = w0v[r]
                wr1 = w1v[r]
                for k in range(DM // 16):
                    sl = pl.ds(k * 16, 16)
                    obuf[r, sl] = wr0 * y0buf[r, sl] + wr1 * y1buf[r, sl]
                return 0

            lax.fori_loop(0, 16, _row, 0)
            pltpu.sync_copy(obuf, out_hbm.at[pl.ds(cb, 16)])

    return _combine_k(ys, p0, p1, w0b, w1b)


def kernel(x, router_w, w1, w2):
    xf = x.reshape(-1, DM)
    m_rows = xf.shape[0]
    rmax = NTOPK * m_rows + NE * T
    rwt = router_w.T

    tw0, tw1, w0b, w1b, p0, p1, te = _routing_call(xf, rwt)
    p0f = p0.reshape(-1)
    p1f = p1.reshape(-1)

    xs = _scatter_call(xf, p0f, p1f, rmax)
    ys = _gemm_call(xs, te.reshape(-1), w1, w2)
    out = _combine_call(ys, p0f, p1f, w0b, w1b)

    topk_weights = jnp.concatenate([tw0, tw1], axis=1)
    return out.reshape(x.shape), topk_weights
